# baseline (speedup 1.0000x reference)
import functools

import jax
import jax.numpy as jnp
from jax import lax
from jax.experimental import pallas as pl
from jax.experimental.pallas import tpu as pltpu
from jax.experimental.pallas import tpu_sc as plsc

NUM_EMB = 1000000
DIM = 32
B = 16384

_NC = 2
_NS = 16
_NW = _NC * _NS
_BPW = B // _NW

_mesh = plsc.VectorSubcoreMesh(core_axis_name="c", subcore_axis_name="s")


@functools.partial(
    pl.kernel,
    mesh=_mesh,
    out_type=jax.ShapeDtypeStruct((2, B, DIM), jnp.float32),
    scratch_types=[
        pltpu.VMEM((_BPW,), jnp.int32),
        pltpu.SemaphoreType.DMA,
    ],
)
def _emb_lookup(idx_hbm, wr_hbm, wi_hbm, out_hbm, idx_v, sem):
    wid = lax.axis_index("s") * _NC + lax.axis_index("c")
    base = wid * _BPW
    pltpu.sync_copy(idx_hbm.at[wid], idx_v)

    def body(g, _):
        grp = idx_v[pl.ds(g * 16, 16)]
        for lane in range(16):
            row = grp[lane]
            i = g * 16 + lane
            pltpu.async_copy(
                wr_hbm.at[pl.ds(row, 1), :], out_hbm.at[0, pl.ds(base + i, 1), :], sem)
            pltpu.async_copy(
                wi_hbm.at[pl.ds(row, 1), :], out_hbm.at[1, pl.ds(base + i, 1), :], sem)
        return ()

    lax.fori_loop(0, _BPW // 16, body, ())
    pltpu.make_async_copy(
        wr_hbm.at[pl.ds(0, _BPW), :], out_hbm.at[0, pl.ds(base, _BPW), :], sem).wait()
    pltpu.make_async_copy(
        wi_hbm.at[pl.ds(0, _BPW), :], out_hbm.at[1, pl.ds(base, _BPW), :], sem).wait()


@jax.jit
def kernel(index, W_real, W_img):
    idx = index.astype(jnp.int32).reshape(_NW, _BPW)
    return _emb_lookup(idx, W_real, W_img)


# speedup vs baseline: 1.8114x; 1.8114x over previous
import functools

import jax
import jax.numpy as jnp
from jax import lax
from jax.experimental import pallas as pl
from jax.experimental.pallas import tpu as pltpu
from jax.experimental.pallas import tpu_sc as plsc

NUM_EMB = 1000000
DIM = 32
B = 16384

_NC = 2
_NS = 16
_NW = _NC * _NS
_BPW = B // _NW

_mesh = plsc.VectorSubcoreMesh(core_axis_name="c", subcore_axis_name="s")


@functools.partial(
    pl.kernel,
    mesh=_mesh,
    out_type=jax.ShapeDtypeStruct((2, B, DIM), jnp.float32),
    scratch_types=[
        pltpu.VMEM((_BPW,), jnp.int32),
        pltpu.VMEM((_BPW, DIM), jnp.float32),
        pltpu.SemaphoreType.DMA,
    ],
)
def _emb_lookup(idx_hbm, wr_hbm, wi_hbm, out_hbm, idx_v, rows, sem):
    wid = lax.axis_index("s") * _NC + lax.axis_index("c")
    base = wid * _BPW
    pltpu.sync_copy(idx_hbm.at[wid], idx_v)

    for t, w_hbm in ((0, wr_hbm), (1, wi_hbm)):
        def grp_body(g, _):
            grp = idx_v[pl.ds(g * 16, 16)]
            for lane in range(16):
                row = grp[lane]
                pltpu.async_copy(
                    w_hbm.at[pl.ds(row, 1), :],
                    rows.at[pl.ds(g * 16 + lane, 1), :],
                    sem)
            return ()

        lax.fori_loop(0, _BPW // 16, grp_body, ())
        pltpu.make_async_copy(w_hbm.at[pl.ds(0, _BPW), :], rows, sem).wait()
        pltpu.sync_copy(rows, out_hbm.at[t, pl.ds(base, _BPW), :])


@jax.jit
def kernel(index, W_real, W_img):
    idx = index.astype(jnp.int32).reshape(_NW, _BPW)
    return _emb_lookup(idx, W_real, W_img)
